# baseline (speedup 1.0000x reference)
import functools

import jax
import jax.numpy as jnp
from jax import lax
from jax.experimental import pallas as pl
from jax.experimental.pallas import tpu as pltpu
from jax.experimental.pallas import tpu_sc as plsc

_TILE = 256


def _knn(unknown_ref, known_ref, idx_ref, w_ref):
    U = unknown_ref[0]
    K = known_ref[0]
    su2 = jnp.sum(U * U, axis=1, keepdims=True)
    sk2 = jnp.sum(K * K, axis=1)[None, :]
    dot = lax.dot_general(U, K, (((1,), (1,)), ((), ())),
                          preferred_element_type=jnp.float32)
    d2 = su2 - 2.0 * dot + sk2
    tile, m = d2.shape
    colf = lax.broadcasted_iota(jnp.int32, (m, 1), 0).astype(jnp.float32)
    BIG = jnp.float32(3.0e38)
    m1 = jnp.min(d2, axis=1, keepdims=True)
    eq1 = (d2 == m1).astype(jnp.float32)
    m2 = jnp.min(jnp.where(d2 <= m1, BIG, d2), axis=1, keepdims=True)
    eq2 = (d2 == m2).astype(jnp.float32)
    m3 = jnp.min(jnp.where(d2 <= m2, BIG, d2), axis=1, keepdims=True)
    eq3 = (d2 == m3).astype(jnp.float32)
    stacked = jnp.concatenate([eq1, eq2, eq3], axis=0)
    fidx = lax.dot_general(stacked, colf, (((1,), (0,)), ((), ())),
                           preferred_element_type=jnp.float32)
    i123 = jnp.concatenate(
        [fidx[0:tile], fidx[tile:2 * tile], fidx[2 * tile:3 * tile]], axis=1)
    r1 = 1.0 / (m1 + 1e-8)
    r2 = 1.0 / (m2 + 1e-8)
    r3 = 1.0 / (m3 + 1e-8)
    norm = r1 + r2 + r3
    idx_ref[0] = i123.astype(jnp.int32)
    w_ref[0] = jnp.concatenate([r1 / norm, r2 / norm, r3 / norm], axis=1)


def _make_sc_interp(B, n, m, C2):
    info = plsc.get_sparse_core_info()
    NC, NS, L = info.num_cores, info.num_subcores, info.num_lanes
    ncc = C2 // L
    PC = 512

    @functools.partial(
        pl.kernel,
        mesh=plsc.VectorSubcoreMesh(core_axis_name="c", subcore_axis_name="s"),
        compiler_params=pltpu.CompilerParams(
            use_tc_tiling_on_sc=False, needs_layout_passes=False),
        out_type=jax.ShapeDtypeStruct((B, C2, n), jnp.float32),
        scratch_types=[
            pltpu.VMEM((PC, 3), jnp.int32),
            pltpu.VMEM((PC, 3), jnp.float32),
            pltpu.VMEM((m, L), jnp.float32),
            pltpu.VMEM((L, PC), jnp.float32),
        ],
    )
    def sc_interp(idx_hbm, w_hbm, kf_hbm, out_hbm, ic_v, wc_v, tab_v, ob_v):
        wid = lax.axis_index("s") * NC + lax.axis_index("c")
        b = wid // ncc
        cc = wid % ncc
        pltpu.sync_copy(kf_hbm.at[b, cc], tab_v)
        iota = lax.iota(jnp.int32, L)
        nchunks = n // PC
        ngroups = PC // L

        def chunk_body(ch, carry):
            pltpu.sync_copy(idx_hbm.at[b, pl.ds(ch * PC, PC)], ic_v)
            pltpu.sync_copy(w_hbm.at[b, pl.ds(ch * PC, PC)], wc_v)

            def group_body(g, carry2):
                rows = g * L + iota
                accs = None
                for k in range(3):
                    kv = jnp.full((L,), k, jnp.int32)
                    i16 = plsc.load_gather(ic_v, [rows, kv])
                    w16 = plsc.load_gather(wc_v, [rows, kv])
                    cur = []
                    for c in range(L):
                        cv = jnp.full((L,), c, jnp.int32)
                        g16 = plsc.load_gather(tab_v, [i16, cv])
                        contrib = w16 * g16
                        cur.append(contrib if accs is None
                                   else accs[c] + contrib)
                    accs = cur
                for c, acc in enumerate(accs):
                    plsc.store_scatter(
                        ob_v, [jnp.full((L,), c, jnp.int32), rows], acc)
                return carry2

            lax.fori_loop(0, ngroups, group_body, 0)
            pltpu.sync_copy(
                ob_v, out_hbm.at[b, pl.ds(cc * L, L), pl.ds(ch * PC, PC)])
            return carry

        lax.fori_loop(0, nchunks, chunk_body, 0)

    return sc_interp


def _bn_scale_shift(s_ref, q_ref, g_ref, be_ref, N):
    mu = s_ref[...] / N
    var = q_ref[...] / N - mu * mu
    a = g_ref[...] * lax.rsqrt(var + 1e-5)
    c = be_ref[...] - a * mu
    return a, c


def _mlp(interp_ref, uf_ref, W1_ref, b1_ref, W2_ref, b2_ref,
         g1_ref, be1_ref, g2_ref, be2_ref, out_ref,
         h1_ref, h2_ref, s1_ref, q1_ref, s2_ref, q2_ref, *, NT, N):
    p = pl.program_id(0)
    b = pl.program_id(1)
    t = pl.program_id(2)
    tile = out_ref.shape[2]
    col0 = (b * NT + t) * tile
    first = jnp.logical_and(b == 0, t == 0)

    @pl.when(p == 0)
    def _phase0():
        feat = jnp.concatenate([interp_ref[0], uf_ref[0]], axis=0)
        h1 = lax.dot_general(W1_ref[...], feat, (((1,), (0,)), ((), ())),
                             preferred_element_type=jnp.float32) + b1_ref[...]
        h1_ref[:, pl.ds(col0, tile)] = h1

        @pl.when(first)
        def _():
            s1_ref[...] = jnp.zeros_like(s1_ref)
            q1_ref[...] = jnp.zeros_like(q1_ref)

        s1_ref[...] += jnp.sum(h1, axis=1, keepdims=True)
        q1_ref[...] += jnp.sum(h1 * h1, axis=1, keepdims=True)

    @pl.when(p == 1)
    def _phase1():
        a1, c1 = _bn_scale_shift(s1_ref, q1_ref, g1_ref, be1_ref, N)
        r = jnp.maximum(a1 * h1_ref[:, pl.ds(col0, tile)] + c1, 0.0)
        h2 = lax.dot_general(W2_ref[...], r, (((1,), (0,)), ((), ())),
                             preferred_element_type=jnp.float32) + b2_ref[...]
        h2_ref[:, pl.ds(col0, tile)] = h2

        @pl.when(first)
        def _():
            s2_ref[...] = jnp.zeros_like(s2_ref)
            q2_ref[...] = jnp.zeros_like(q2_ref)

        s2_ref[...] += jnp.sum(h2, axis=1, keepdims=True)
        q2_ref[...] += jnp.sum(h2 * h2, axis=1, keepdims=True)

    @pl.when(p == 2)
    def _phase2():
        a2, c2 = _bn_scale_shift(s2_ref, q2_ref, g2_ref, be2_ref, N)
        out_ref[0] = jnp.maximum(a2 * h2_ref[:, pl.ds(col0, tile)] + c2, 0.0)


def kernel(unknown, known, unknow_feats, known_feats,
           W1, b1, g1, be1, W2, b2, g2, be2):
    B, n, _ = unknown.shape
    m = known.shape[1]
    C1 = unknow_feats.shape[1]
    C2 = known_feats.shape[1]
    Cin = C1 + C2
    Co = W1.shape[0]
    tile = min(_TILE, n)
    NT = n // tile
    N = float(B * n)

    kfT = jnp.transpose(known_feats, (0, 2, 1))
    b1c, g1c, be1c = b1[:, None], g1[:, None], be1[:, None]
    b2c, g2c, be2c = b2[:, None], g2[:, None], be2[:, None]

    idx, w = pl.pallas_call(
        _knn,
        grid=(B, NT),
        in_specs=[
            pl.BlockSpec((1, tile, 3), lambda b, t: (b, t, 0)),
            pl.BlockSpec((1, m, 3), lambda b, t: (b, 0, 0)),
        ],
        out_specs=[
            pl.BlockSpec((1, tile, 3), lambda b, t: (b, t, 0)),
            pl.BlockSpec((1, tile, 3), lambda b, t: (b, t, 0)),
        ],
        out_shape=[
            jax.ShapeDtypeStruct((B, n, 3), jnp.int32),
            jax.ShapeDtypeStruct((B, n, 3), jnp.float32),
        ],
    )(unknown, known)

    L = plsc.get_sparse_core_info().num_lanes
    ncc = C2 // L
    kf4 = jnp.transpose(known_feats.reshape(B, ncc, L, m), (0, 1, 3, 2))
    interpT = _make_sc_interp(B, n, m, C2)(idx, w, kf4)

    vec = pl.BlockSpec((Co, 1), lambda p, b, t: (0, 0))
    out = pl.pallas_call(
        functools.partial(_mlp, NT=NT, N=N),
        grid=(3, B, NT),
        in_specs=[
            pl.BlockSpec((1, C2, tile),
                         lambda p, b, t: ((p == 0) * b, 0, (p == 0) * t)),
            pl.BlockSpec((1, C1, tile),
                         lambda p, b, t: ((p == 0) * b, 0, (p == 0) * t)),
            pl.BlockSpec((Co, Cin), lambda p, b, t: (0, 0)),
            vec,
            pl.BlockSpec((Co, Co), lambda p, b, t: (0, 0)),
            vec, vec, vec, vec, vec,
        ],
        out_specs=pl.BlockSpec((1, Co, tile),
                               lambda p, b, t: ((p == 2) * b, 0, (p == 2) * t)),
        out_shape=jax.ShapeDtypeStruct((B, Co, n), jnp.float32),
        scratch_shapes=[
            pltpu.VMEM((Co, B * n), jnp.float32),
            pltpu.VMEM((Co, B * n), jnp.float32),
            pltpu.VMEM((Co, 1), jnp.float32),
            pltpu.VMEM((Co, 1), jnp.float32),
            pltpu.VMEM((Co, 1), jnp.float32),
            pltpu.VMEM((Co, 1), jnp.float32),
        ],
    )(interpT, unknow_feats, W1, b1c, W2, b2c, g1c, be1c, g2c, be2c)

    return out


# speedup vs baseline: 1.1267x; 1.1267x over previous
import functools

import jax
import jax.numpy as jnp
from jax import lax
from jax.experimental import pallas as pl
from jax.experimental.pallas import tpu as pltpu
from jax.experimental.pallas import tpu_sc as plsc

_TILE = 256


def _knn(unknown_ref, known_ref, idx_ref, w_ref):
    U = unknown_ref[0]
    K = known_ref[0]
    su2 = jnp.sum(U * U, axis=1, keepdims=True)
    sk2 = jnp.sum(K * K, axis=1)[None, :]
    dot = lax.dot_general(U, K, (((1,), (1,)), ((), ())),
                          preferred_element_type=jnp.float32)
    d2 = su2 - 2.0 * dot + sk2
    tile, m = d2.shape
    col = lax.broadcasted_iota(jnp.int32, (tile, m), 1)
    BIG = jnp.float32(3.0e38)
    IBIG = jnp.int32(2147483647)
    m1 = jnp.min(d2, axis=1, keepdims=True)
    i1 = jnp.min(jnp.where(d2 == m1, col, IBIG), axis=1, keepdims=True)
    d2b = jnp.where(d2 <= m1, BIG, d2)
    m2 = jnp.min(d2b, axis=1, keepdims=True)
    i2 = jnp.min(jnp.where(d2b == m2, col, IBIG), axis=1, keepdims=True)
    d2c = jnp.where(d2 <= m2, BIG, d2)
    m3 = jnp.min(d2c, axis=1, keepdims=True)
    i3 = jnp.min(jnp.where(d2c == m3, col, IBIG), axis=1, keepdims=True)
    r1 = 1.0 / (m1 + 1e-8)
    r2 = 1.0 / (m2 + 1e-8)
    r3 = 1.0 / (m3 + 1e-8)
    norm = r1 + r2 + r3
    idx_ref[0] = jnp.concatenate([i1, i2, i3], axis=1)
    w_ref[0] = jnp.concatenate([r1 / norm, r2 / norm, r3 / norm], axis=1)


def _make_sc_interp(B, n, m, C2):
    info = plsc.get_sparse_core_info()
    NC, NS, L = info.num_cores, info.num_subcores, info.num_lanes
    ncc = C2 // L
    PC = 512

    NW = NC * NS
    nps = NW // (B * ncc)
    npts = n // max(nps, 1)

    @functools.partial(
        pl.kernel,
        mesh=plsc.VectorSubcoreMesh(core_axis_name="c", subcore_axis_name="s"),
        compiler_params=pltpu.CompilerParams(
            use_tc_tiling_on_sc=False, needs_layout_passes=False),
        out_type=jax.ShapeDtypeStruct((B, C2, n), jnp.float32),
        scratch_types=[
            pltpu.VMEM((PC, 3), jnp.int32),
            pltpu.VMEM((PC, 3), jnp.float32),
            pltpu.VMEM((m, L), jnp.float32),
            pltpu.VMEM((L, PC), jnp.float32),
        ],
    )
    def sc_interp(idx_hbm, w_hbm, kf_hbm, out_hbm, ic_v, wc_v, tab_v, ob_v):
        wid = lax.axis_index("s") * NC + lax.axis_index("c")
        b = wid // (ncc * nps)
        cc = (wid // nps) % ncc
        ps = wid % nps
        p0 = ps * npts
        pltpu.sync_copy(kf_hbm.at[b, cc], tab_v)
        iota = lax.iota(jnp.int32, L)
        nchunks = npts // PC
        ngroups = PC // L

        def chunk_body(ch, carry):
            pltpu.sync_copy(idx_hbm.at[b, pl.ds(p0 + ch * PC, PC)], ic_v)
            pltpu.sync_copy(w_hbm.at[b, pl.ds(p0 + ch * PC, PC)], wc_v)

            def group_body(g, carry2):
                rows = g * L + iota
                accs = None
                for k in range(3):
                    kv = jnp.full((L,), k, jnp.int32)
                    i16 = plsc.load_gather(ic_v, [rows, kv])
                    w16 = plsc.load_gather(wc_v, [rows, kv])
                    cur = []
                    for c in range(L):
                        cv = jnp.full((L,), c, jnp.int32)
                        g16 = plsc.load_gather(tab_v, [i16, cv])
                        contrib = w16 * g16
                        cur.append(contrib if accs is None
                                   else accs[c] + contrib)
                    accs = cur
                for c, acc in enumerate(accs):
                    plsc.store_scatter(
                        ob_v, [jnp.full((L,), c, jnp.int32), rows], acc)
                return carry2

            lax.fori_loop(0, ngroups, group_body, 0)
            pltpu.sync_copy(
                ob_v,
                out_hbm.at[b, pl.ds(cc * L, L), pl.ds(p0 + ch * PC, PC)])
            return carry

        lax.fori_loop(0, nchunks, chunk_body, 0)

    return sc_interp


def _bn_scale_shift(s_ref, q_ref, g_ref, be_ref, N):
    mu = s_ref[...] / N
    var = q_ref[...] / N - mu * mu
    a = g_ref[...] * lax.rsqrt(var + 1e-5)
    c = be_ref[...] - a * mu
    return a, c


def _mlp(interpa_ref, interpb_ref, uf_ref, W1_ref, b1_ref, W2_ref, b2_ref,
         g1_ref, be1_ref, g2_ref, be2_ref, out_ref,
         h1_ref, h2_ref, s1_ref, q1_ref, s2_ref, q2_ref, *, NT, N, Bh):
    p = pl.program_id(0)
    b = pl.program_id(1)
    t = pl.program_id(2)
    tile = out_ref.shape[2]
    col0 = (b * NT + t) * tile
    first = jnp.logical_and(b == 0, t == 0)

    @pl.when(p == 0)
    def _phase0():
        interp = jnp.where(b < Bh, interpa_ref[0], interpb_ref[0])
        feat = jnp.concatenate([interp, uf_ref[0]], axis=0)
        h1 = lax.dot_general(W1_ref[...], feat, (((1,), (0,)), ((), ())),
                             preferred_element_type=jnp.float32) + b1_ref[...]
        h1_ref[:, pl.ds(col0, tile)] = h1

        @pl.when(first)
        def _():
            s1_ref[...] = jnp.zeros_like(s1_ref)
            q1_ref[...] = jnp.zeros_like(q1_ref)

        s1_ref[...] += jnp.sum(h1, axis=1, keepdims=True)
        q1_ref[...] += jnp.sum(h1 * h1, axis=1, keepdims=True)

    @pl.when(p == 1)
    def _phase1():
        a1, c1 = _bn_scale_shift(s1_ref, q1_ref, g1_ref, be1_ref, N)
        r = jnp.maximum(a1 * h1_ref[:, pl.ds(col0, tile)] + c1, 0.0)
        h2 = lax.dot_general(W2_ref[...], r, (((1,), (0,)), ((), ())),
                             preferred_element_type=jnp.float32) + b2_ref[...]
        h2_ref[:, pl.ds(col0, tile)] = h2

        @pl.when(first)
        def _():
            s2_ref[...] = jnp.zeros_like(s2_ref)
            q2_ref[...] = jnp.zeros_like(q2_ref)

        s2_ref[...] += jnp.sum(h2, axis=1, keepdims=True)
        q2_ref[...] += jnp.sum(h2 * h2, axis=1, keepdims=True)

    @pl.when(p == 2)
    def _phase2():
        a2, c2 = _bn_scale_shift(s2_ref, q2_ref, g2_ref, be2_ref, N)
        out_ref[0] = jnp.maximum(a2 * h2_ref[:, pl.ds(col0, tile)] + c2, 0.0)


def kernel(unknown, known, unknow_feats, known_feats,
           W1, b1, g1, be1, W2, b2, g2, be2):
    B, n, _ = unknown.shape
    m = known.shape[1]
    C1 = unknow_feats.shape[1]
    C2 = known_feats.shape[1]
    Cin = C1 + C2
    Co = W1.shape[0]
    tile = min(_TILE, n)
    NT = n // tile
    N = float(B * n)

    kfT = jnp.transpose(known_feats, (0, 2, 1))
    b1c, g1c, be1c = b1[:, None], g1[:, None], be1[:, None]
    b2c, g2c, be2c = b2[:, None], g2[:, None], be2[:, None]

    L = plsc.get_sparse_core_info().num_lanes
    ncc = C2 // L
    kf4 = jnp.transpose(known_feats.reshape(B, ncc, L, m), (0, 1, 3, 2))

    nh = 2 if (B % 2 == 0 and B >= 2) else 1
    Bh = B // nh
    sc_call = _make_sc_interp(Bh, n, m, C2)

    def knn_half(u_h, k_h):
        return pl.pallas_call(
            _knn,
            grid=(Bh, NT),
            in_specs=[
                pl.BlockSpec((1, tile, 3), lambda b, t: (b, t, 0)),
                pl.BlockSpec((1, m, 3), lambda b, t: (b, 0, 0)),
            ],
            out_specs=[
                pl.BlockSpec((1, tile, 3), lambda b, t: (b, t, 0)),
                pl.BlockSpec((1, tile, 3), lambda b, t: (b, t, 0)),
            ],
            out_shape=[
                jax.ShapeDtypeStruct((Bh, n, 3), jnp.int32),
                jax.ShapeDtypeStruct((Bh, n, 3), jnp.float32),
            ],
        )(u_h, k_h)

    interps = []
    for h in range(nh):
        sl = slice(h * Bh, (h + 1) * Bh)
        idx_h, w_h = knn_half(unknown[sl], known[sl])
        interps.append(sc_call(idx_h, w_h, kf4[sl]))
    interp_a = interps[0]
    interp_b = interps[-1]

    vec = pl.BlockSpec((Co, 1), lambda p, b, t: (0, 0))
    out = pl.pallas_call(
        functools.partial(_mlp, NT=NT, N=N, Bh=Bh),
        grid=(3, B, NT),
        in_specs=[
            pl.BlockSpec(
                (1, C2, tile),
                lambda p, b, t: ((p == 0) * jnp.minimum(b, Bh - 1), 0,
                                 (p == 0) * t)),
            pl.BlockSpec(
                (1, C2, tile),
                lambda p, b, t: ((p == 0) * jnp.maximum(b - Bh, 0), 0,
                                 (p == 0) * t)),
            pl.BlockSpec((1, C1, tile),
                         lambda p, b, t: ((p == 0) * b, 0, (p == 0) * t)),
            pl.BlockSpec((Co, Cin), lambda p, b, t: (0, 0)),
            vec,
            pl.BlockSpec((Co, Co), lambda p, b, t: (0, 0)),
            vec, vec, vec, vec, vec,
        ],
        out_specs=pl.BlockSpec((1, Co, tile),
                               lambda p, b, t: ((p == 2) * b, 0, (p == 2) * t)),
        out_shape=jax.ShapeDtypeStruct((B, Co, n), jnp.float32),
        scratch_shapes=[
            pltpu.VMEM((Co, B * n), jnp.float32),
            pltpu.VMEM((Co, B * n), jnp.float32),
            pltpu.VMEM((Co, 1), jnp.float32),
            pltpu.VMEM((Co, 1), jnp.float32),
            pltpu.VMEM((Co, 1), jnp.float32),
            pltpu.VMEM((Co, 1), jnp.float32),
        ],
    )(interp_a, interp_b, unknow_feats, W1, b1c, W2, b2c,
      g1c, be1c, g2c, be2c)

    return out


# speedup vs baseline: 1.1484x; 1.0192x over previous
import functools

import jax
import jax.numpy as jnp
from jax import lax
from jax.experimental import pallas as pl
from jax.experimental.pallas import tpu as pltpu
from jax.experimental.pallas import tpu_sc as plsc

_TILE = 256


def _knn(unknown_ref, known_ref, idx_ref, w_ref):
    U = unknown_ref[0]
    K = known_ref[0]
    su2 = jnp.sum(U * U, axis=1, keepdims=True)
    sk2 = jnp.sum(K * K, axis=1)[None, :]
    dot = lax.dot_general(U, K, (((1,), (1,)), ((), ())),
                          preferred_element_type=jnp.float32)
    d2 = su2 - 2.0 * dot + sk2
    tile, m = d2.shape
    col = lax.broadcasted_iota(jnp.int32, (tile, m), 1)
    BIG = jnp.float32(3.0e38)
    IBIG = jnp.int32(2147483647)
    m1 = jnp.min(d2, axis=1, keepdims=True)
    i1 = jnp.min(jnp.where(d2 == m1, col, IBIG), axis=1, keepdims=True)
    d2b = jnp.where(d2 <= m1, BIG, d2)
    m2 = jnp.min(d2b, axis=1, keepdims=True)
    i2 = jnp.min(jnp.where(d2b == m2, col, IBIG), axis=1, keepdims=True)
    d2c = jnp.where(d2 <= m2, BIG, d2)
    m3 = jnp.min(d2c, axis=1, keepdims=True)
    i3 = jnp.min(jnp.where(d2c == m3, col, IBIG), axis=1, keepdims=True)
    r1 = 1.0 / (m1 + 1e-8)
    r2 = 1.0 / (m2 + 1e-8)
    r3 = 1.0 / (m3 + 1e-8)
    norm = r1 + r2 + r3
    idx_ref[0] = jnp.concatenate([i1, i2, i3], axis=1)
    w_ref[0] = jnp.concatenate([r1 / norm, r2 / norm, r3 / norm], axis=1)


def _make_sc_interp(B, n, m, C2):
    info = plsc.get_sparse_core_info()
    NC, NS, L = info.num_cores, info.num_subcores, info.num_lanes
    ncc = C2 // L
    PC = 512

    NW = NC * NS
    nps = NW // (B * ncc)
    npts = n // max(nps, 1)

    nchunks = npts // PC
    ngroups = PC // L

    @functools.partial(
        pl.kernel,
        mesh=plsc.VectorSubcoreMesh(core_axis_name="c", subcore_axis_name="s"),
        compiler_params=pltpu.CompilerParams(
            use_tc_tiling_on_sc=False, needs_layout_passes=False),
        out_type=jax.ShapeDtypeStruct((B, C2, n), jnp.float32),
        scratch_types=[
            pltpu.VMEM((PC, 3), jnp.int32),
            pltpu.VMEM((PC, 3), jnp.int32),
            pltpu.VMEM((PC, 3), jnp.float32),
            pltpu.VMEM((PC, 3), jnp.float32),
            pltpu.VMEM((m, L), jnp.float32),
            pltpu.VMEM((L, PC), jnp.float32),
            pltpu.VMEM((L, PC), jnp.float32),
            pltpu.SemaphoreType.DMA,
            pltpu.SemaphoreType.DMA,
            pltpu.SemaphoreType.DMA,
            pltpu.SemaphoreType.DMA,
        ],
    )
    def sc_interp(idx_hbm, w_hbm, kf_hbm, out_hbm,
                  ic0, ic1, wc0, wc1, tab_v, ob0, ob1,
                  si0, si1, so0, so1):
        wid = lax.axis_index("s") * NC + lax.axis_index("c")
        b = wid // (ncc * nps)
        cc = (wid // nps) % ncc
        ps = wid % nps
        p0 = ps * npts
        pltpu.sync_copy(kf_hbm.at[b, cc], tab_v)
        iota = lax.iota(jnp.int32, L)
        bufs = [(ic0, wc0, ob0, si0, so0), (ic1, wc1, ob1, si1, so1)]

        def in_descs(ch, ic, wc, si):
            src = pl.ds(p0 + ch * PC, PC)
            return (pltpu.make_async_copy(idx_hbm.at[b, src], ic, si),
                    pltpu.make_async_copy(w_hbm.at[b, src], wc, si))

        def out_desc(ch, ob, so):
            return pltpu.make_async_copy(
                ob, out_hbm.at[b, pl.ds(cc * L, L), pl.ds(p0 + ch * PC, PC)],
                so)

        for d in in_descs(0, *bufs[0][:2], bufs[0][3]):
            d.start()
        for ch in range(nchunks):
            ic, wc, ob, si, so = bufs[ch % 2]
            if ch + 1 < nchunks:
                nic, nwc, _, nsi, _ = bufs[(ch + 1) % 2]
                for d in in_descs(ch + 1, nic, nwc, nsi):
                    d.start()
            for d in in_descs(ch, ic, wc, si):
                d.wait()
            if ch >= 2:
                out_desc(ch - 2, ob, so).wait()

            def group_body(g, carry, ic=ic, wc=wc, ob=ob):
                rows = g * L + iota
                accs = None
                for k in range(3):
                    kv = jnp.full((L,), k, jnp.int32)
                    i16 = plsc.load_gather(ic, [rows, kv])
                    w16 = plsc.load_gather(wc, [rows, kv])
                    cur = []
                    for c in range(L):
                        cv = jnp.full((L,), c, jnp.int32)
                        g16 = plsc.load_gather(tab_v, [i16, cv])
                        contrib = w16 * g16
                        cur.append(contrib if accs is None
                                   else accs[c] + contrib)
                    accs = cur
                for c, acc in enumerate(accs):
                    plsc.store_scatter(
                        ob, [jnp.full((L,), c, jnp.int32), rows], acc)
                return carry

            lax.fori_loop(0, ngroups, group_body, 0)
            out_desc(ch, ob, so).start()
        for ch in range(max(nchunks - 2, 0), nchunks):
            _, _, ob, _, so = bufs[ch % 2]
            out_desc(ch, ob, so).wait()

    return sc_interp


def _bn_scale_shift(s_ref, q_ref, g_ref, be_ref, N):
    mu = s_ref[...] / N
    var = q_ref[...] / N - mu * mu
    a = g_ref[...] * lax.rsqrt(var + 1e-5)
    c = be_ref[...] - a * mu
    return a, c


def _mlp(interpa_ref, interpb_ref, uf_ref, W1_ref, b1_ref, W2_ref, b2_ref,
         g1_ref, be1_ref, g2_ref, be2_ref, out_ref,
         h1_ref, h2_ref, s1_ref, q1_ref, s2_ref, q2_ref, *, NT, N, Bh):
    p = pl.program_id(0)
    b = pl.program_id(1)
    t = pl.program_id(2)
    tile = out_ref.shape[2]
    col0 = (b * NT + t) * tile
    first = jnp.logical_and(b == 0, t == 0)

    @pl.when(p == 0)
    def _phase0():
        interp = jnp.where(b < Bh, interpa_ref[0], interpb_ref[0])
        feat = jnp.concatenate([interp, uf_ref[0]], axis=0)
        h1 = lax.dot_general(W1_ref[...], feat, (((1,), (0,)), ((), ())),
                             preferred_element_type=jnp.float32) + b1_ref[...]
        h1_ref[:, pl.ds(col0, tile)] = h1

        @pl.when(first)
        def _():
            s1_ref[...] = jnp.zeros_like(s1_ref)
            q1_ref[...] = jnp.zeros_like(q1_ref)

        s1_ref[...] += jnp.sum(h1, axis=1, keepdims=True)
        q1_ref[...] += jnp.sum(h1 * h1, axis=1, keepdims=True)

    @pl.when(p == 1)
    def _phase1():
        a1, c1 = _bn_scale_shift(s1_ref, q1_ref, g1_ref, be1_ref, N)
        r = jnp.maximum(a1 * h1_ref[:, pl.ds(col0, tile)] + c1, 0.0)
        h2 = lax.dot_general(W2_ref[...], r, (((1,), (0,)), ((), ())),
                             preferred_element_type=jnp.float32) + b2_ref[...]
        h2_ref[:, pl.ds(col0, tile)] = h2

        @pl.when(first)
        def _():
            s2_ref[...] = jnp.zeros_like(s2_ref)
            q2_ref[...] = jnp.zeros_like(q2_ref)

        s2_ref[...] += jnp.sum(h2, axis=1, keepdims=True)
        q2_ref[...] += jnp.sum(h2 * h2, axis=1, keepdims=True)

    @pl.when(p == 2)
    def _phase2():
        a2, c2 = _bn_scale_shift(s2_ref, q2_ref, g2_ref, be2_ref, N)
        out_ref[0] = jnp.maximum(a2 * h2_ref[:, pl.ds(col0, tile)] + c2, 0.0)


def kernel(unknown, known, unknow_feats, known_feats,
           W1, b1, g1, be1, W2, b2, g2, be2):
    B, n, _ = unknown.shape
    m = known.shape[1]
    C1 = unknow_feats.shape[1]
    C2 = known_feats.shape[1]
    Cin = C1 + C2
    Co = W1.shape[0]
    tile = min(_TILE, n)
    NT = n // tile
    N = float(B * n)

    kfT = jnp.transpose(known_feats, (0, 2, 1))
    b1c, g1c, be1c = b1[:, None], g1[:, None], be1[:, None]
    b2c, g2c, be2c = b2[:, None], g2[:, None], be2[:, None]

    L = plsc.get_sparse_core_info().num_lanes
    ncc = C2 // L
    kf4 = jnp.transpose(known_feats.reshape(B, ncc, L, m), (0, 1, 3, 2))

    nh = 2 if (B % 2 == 0 and B >= 2) else 1
    Bh = B // nh
    sc_call = _make_sc_interp(Bh, n, m, C2)

    def knn_half(u_h, k_h):
        return pl.pallas_call(
            _knn,
            grid=(Bh, NT),
            in_specs=[
                pl.BlockSpec((1, tile, 3), lambda b, t: (b, t, 0)),
                pl.BlockSpec((1, m, 3), lambda b, t: (b, 0, 0)),
            ],
            out_specs=[
                pl.BlockSpec((1, tile, 3), lambda b, t: (b, t, 0)),
                pl.BlockSpec((1, tile, 3), lambda b, t: (b, t, 0)),
            ],
            out_shape=[
                jax.ShapeDtypeStruct((Bh, n, 3), jnp.int32),
                jax.ShapeDtypeStruct((Bh, n, 3), jnp.float32),
            ],
        )(u_h, k_h)

    interps = []
    for h in range(nh):
        sl = slice(h * Bh, (h + 1) * Bh)
        idx_h, w_h = knn_half(unknown[sl], known[sl])
        interps.append(sc_call(idx_h, w_h, kf4[sl]))
    interp_a = interps[0]
    interp_b = interps[-1]

    vec = pl.BlockSpec((Co, 1), lambda p, b, t: (0, 0))
    out = pl.pallas_call(
        functools.partial(_mlp, NT=NT, N=N, Bh=Bh),
        grid=(3, B, NT),
        in_specs=[
            pl.BlockSpec(
                (1, C2, tile),
                lambda p, b, t: ((p == 0) * jnp.minimum(b, Bh - 1), 0,
                                 (p == 0) * t)),
            pl.BlockSpec(
                (1, C2, tile),
                lambda p, b, t: ((p == 0) * jnp.maximum(b - Bh, 0), 0,
                                 (p == 0) * t)),
            pl.BlockSpec((1, C1, tile),
                         lambda p, b, t: ((p == 0) * b, 0, (p == 0) * t)),
            pl.BlockSpec((Co, Cin), lambda p, b, t: (0, 0)),
            vec,
            pl.BlockSpec((Co, Co), lambda p, b, t: (0, 0)),
            vec, vec, vec, vec, vec,
        ],
        out_specs=pl.BlockSpec((1, Co, tile),
                               lambda p, b, t: ((p == 2) * b, 0, (p == 2) * t)),
        out_shape=jax.ShapeDtypeStruct((B, Co, n), jnp.float32),
        scratch_shapes=[
            pltpu.VMEM((Co, B * n), jnp.float32),
            pltpu.VMEM((Co, B * n), jnp.float32),
            pltpu.VMEM((Co, 1), jnp.float32),
            pltpu.VMEM((Co, 1), jnp.float32),
            pltpu.VMEM((Co, 1), jnp.float32),
            pltpu.VMEM((Co, 1), jnp.float32),
        ],
    )(interp_a, interp_b, unknow_feats, W1, b1c, W2, b2c,
      g1c, be1c, g2c, be2c)

    return out


# speedup vs baseline: 1.1587x; 1.0089x over previous
import functools

import jax
import jax.numpy as jnp
from jax import lax
from jax.experimental import pallas as pl
from jax.experimental.pallas import tpu as pltpu
from jax.experimental.pallas import tpu_sc as plsc

_TILE = 256


def _knn(unknown_ref, known_ref, idx_ref, w_ref):
    U = unknown_ref[0]
    K = known_ref[0]
    su2 = jnp.sum(U * U, axis=1, keepdims=True)
    sk2 = jnp.sum(K * K, axis=1)[None, :]
    dot = lax.dot_general(U, K, (((1,), (1,)), ((), ())),
                          preferred_element_type=jnp.float32)
    d2 = su2 - 2.0 * dot + sk2
    tile, m = d2.shape
    col = lax.broadcasted_iota(jnp.int32, (tile, m), 1)
    BIG = jnp.float32(3.0e38)
    IBIG = jnp.int32(2147483647)
    m1 = jnp.min(d2, axis=1, keepdims=True)
    i1 = jnp.min(jnp.where(d2 == m1, col, IBIG), axis=1, keepdims=True)
    d2b = jnp.where(d2 <= m1, BIG, d2)
    m2 = jnp.min(d2b, axis=1, keepdims=True)
    i2 = jnp.min(jnp.where(d2b == m2, col, IBIG), axis=1, keepdims=True)
    d2c = jnp.where(d2 <= m2, BIG, d2)
    m3 = jnp.min(d2c, axis=1, keepdims=True)
    i3 = jnp.min(jnp.where(d2c == m3, col, IBIG), axis=1, keepdims=True)
    r1 = 1.0 / (m1 + 1e-8)
    r2 = 1.0 / (m2 + 1e-8)
    r3 = 1.0 / (m3 + 1e-8)
    norm = r1 + r2 + r3
    idx_ref[0] = jnp.concatenate([i1, i2, i3], axis=1)
    w_ref[0] = jnp.concatenate([r1 / norm, r2 / norm, r3 / norm], axis=1)


def _make_sc_interp(B, n, m, C2):
    info = plsc.get_sparse_core_info()
    NC, NS, L = info.num_cores, info.num_subcores, info.num_lanes
    ncc = C2 // L
    PC = 512

    NW = NC * NS
    nps = NW // (B * ncc)
    npts = n // max(nps, 1)

    nchunks = npts // PC
    ngroups = PC // L

    @functools.partial(
        pl.kernel,
        mesh=plsc.VectorSubcoreMesh(core_axis_name="c", subcore_axis_name="s"),
        compiler_params=pltpu.CompilerParams(
            use_tc_tiling_on_sc=False, needs_layout_passes=False),
        out_type=jax.ShapeDtypeStruct((B, C2, n), jnp.float32),
        scratch_types=[
            pltpu.VMEM((PC, 3), jnp.int32),
            pltpu.VMEM((PC, 3), jnp.int32),
            pltpu.VMEM((PC, 3), jnp.float32),
            pltpu.VMEM((PC, 3), jnp.float32),
            pltpu.VMEM((m, L), jnp.float32),
            pltpu.VMEM((L, PC), jnp.float32),
            pltpu.VMEM((L, PC), jnp.float32),
            pltpu.SemaphoreType.DMA,
            pltpu.SemaphoreType.DMA,
            pltpu.SemaphoreType.DMA,
            pltpu.SemaphoreType.DMA,
        ],
    )
    def sc_interp(idx_hbm, w_hbm, kf_hbm, out_hbm,
                  ic0, ic1, wc0, wc1, tab_v, ob0, ob1,
                  si0, si1, so0, so1):
        wid = lax.axis_index("s") * NC + lax.axis_index("c")
        b = wid // (ncc * nps)
        cc = (wid // nps) % ncc
        ps = wid % nps
        p0 = ps * npts
        pltpu.sync_copy(kf_hbm.at[b, cc], tab_v)
        iota = lax.iota(jnp.int32, L)
        bufs = [(ic0, wc0, ob0, si0, so0), (ic1, wc1, ob1, si1, so1)]

        def in_descs(ch, ic, wc, si):
            src = pl.ds(p0 + ch * PC, PC)
            return (pltpu.make_async_copy(idx_hbm.at[b, src], ic, si),
                    pltpu.make_async_copy(w_hbm.at[b, src], wc, si))

        def out_desc(ch, ob, so):
            return pltpu.make_async_copy(
                ob, out_hbm.at[b, pl.ds(cc * L, L), pl.ds(p0 + ch * PC, PC)],
                so)

        for d in in_descs(0, *bufs[0][:2], bufs[0][3]):
            d.start()
        for ch in range(nchunks):
            ic, wc, ob, si, so = bufs[ch % 2]
            if ch + 1 < nchunks:
                nic, nwc, _, nsi, _ = bufs[(ch + 1) % 2]
                for d in in_descs(ch + 1, nic, nwc, nsi):
                    d.start()
            for d in in_descs(ch, ic, wc, si):
                d.wait()
            if ch >= 2:
                out_desc(ch - 2, ob, so).wait()

            def group_body(g2, carry, ic=ic, wc=wc, ob=ob):
                for u in range(2):
                    g = 2 * g2 + u
                    rows = g * L + iota
                    accs = None
                    for k in range(3):
                        kv = jnp.full((L,), k, jnp.int32)
                        i16 = plsc.load_gather(ic, [rows, kv])
                        w16 = plsc.load_gather(wc, [rows, kv])
                        cur = []
                        for c in range(L):
                            cv = jnp.full((L,), c, jnp.int32)
                            g16 = plsc.load_gather(tab_v, [i16, cv])
                            contrib = w16 * g16
                            cur.append(contrib if accs is None
                                       else accs[c] + contrib)
                        accs = cur
                    for c, acc in enumerate(accs):
                        ob[c, pl.ds(g * L, L)] = acc
                return carry

            lax.fori_loop(0, ngroups // 2, group_body, 0)
            out_desc(ch, ob, so).start()
        for ch in range(max(nchunks - 2, 0), nchunks):
            _, _, ob, _, so = bufs[ch % 2]
            out_desc(ch, ob, so).wait()

    return sc_interp


def _bn_scale_shift(s_ref, q_ref, g_ref, be_ref, N):
    mu = s_ref[...] / N
    var = q_ref[...] / N - mu * mu
    a = g_ref[...] * lax.rsqrt(var + 1e-5)
    c = be_ref[...] - a * mu
    return a, c


def _mlp(interpa_ref, interpb_ref, uf_ref, W1_ref, b1_ref, W2_ref, b2_ref,
         g1_ref, be1_ref, g2_ref, be2_ref, out_ref,
         h1_ref, h2_ref, s1_ref, q1_ref, s2_ref, q2_ref, *, NT, N, Bh):
    p = pl.program_id(0)
    b = pl.program_id(1)
    t = pl.program_id(2)
    tile = out_ref.shape[2]
    col0 = (b * NT + t) * tile
    first = jnp.logical_and(b == 0, t == 0)

    @pl.when(p == 0)
    def _phase0():
        interp = jnp.where(b < Bh, interpa_ref[0], interpb_ref[0])
        feat = jnp.concatenate([interp, uf_ref[0]], axis=0)
        h1 = lax.dot_general(W1_ref[...], feat, (((1,), (0,)), ((), ())),
                             preferred_element_type=jnp.float32) + b1_ref[...]
        h1_ref[:, pl.ds(col0, tile)] = h1

        @pl.when(first)
        def _():
            s1_ref[...] = jnp.zeros_like(s1_ref)
            q1_ref[...] = jnp.zeros_like(q1_ref)

        s1_ref[...] += jnp.sum(h1, axis=1, keepdims=True)
        q1_ref[...] += jnp.sum(h1 * h1, axis=1, keepdims=True)

    @pl.when(p == 1)
    def _phase1():
        a1, c1 = _bn_scale_shift(s1_ref, q1_ref, g1_ref, be1_ref, N)
        r = jnp.maximum(a1 * h1_ref[:, pl.ds(col0, tile)] + c1, 0.0)
        h2 = lax.dot_general(W2_ref[...], r, (((1,), (0,)), ((), ())),
                             preferred_element_type=jnp.float32) + b2_ref[...]
        h2_ref[:, pl.ds(col0, tile)] = h2

        @pl.when(first)
        def _():
            s2_ref[...] = jnp.zeros_like(s2_ref)
            q2_ref[...] = jnp.zeros_like(q2_ref)

        s2_ref[...] += jnp.sum(h2, axis=1, keepdims=True)
        q2_ref[...] += jnp.sum(h2 * h2, axis=1, keepdims=True)

    @pl.when(p == 2)
    def _phase2():
        a2, c2 = _bn_scale_shift(s2_ref, q2_ref, g2_ref, be2_ref, N)
        out_ref[0] = jnp.maximum(a2 * h2_ref[:, pl.ds(col0, tile)] + c2, 0.0)


def kernel(unknown, known, unknow_feats, known_feats,
           W1, b1, g1, be1, W2, b2, g2, be2):
    B, n, _ = unknown.shape
    m = known.shape[1]
    C1 = unknow_feats.shape[1]
    C2 = known_feats.shape[1]
    Cin = C1 + C2
    Co = W1.shape[0]
    tile = min(_TILE, n)
    NT = n // tile
    N = float(B * n)

    kfT = jnp.transpose(known_feats, (0, 2, 1))
    b1c, g1c, be1c = b1[:, None], g1[:, None], be1[:, None]
    b2c, g2c, be2c = b2[:, None], g2[:, None], be2[:, None]

    L = plsc.get_sparse_core_info().num_lanes
    ncc = C2 // L
    kf4 = jnp.transpose(known_feats.reshape(B, ncc, L, m), (0, 1, 3, 2))

    nh = 2 if (B % 2 == 0 and B >= 2) else 1
    Bh = B // nh
    sc_call = _make_sc_interp(Bh, n, m, C2)

    def knn_half(u_h, k_h):
        return pl.pallas_call(
            _knn,
            grid=(Bh, NT),
            in_specs=[
                pl.BlockSpec((1, tile, 3), lambda b, t: (b, t, 0)),
                pl.BlockSpec((1, m, 3), lambda b, t: (b, 0, 0)),
            ],
            out_specs=[
                pl.BlockSpec((1, tile, 3), lambda b, t: (b, t, 0)),
                pl.BlockSpec((1, tile, 3), lambda b, t: (b, t, 0)),
            ],
            out_shape=[
                jax.ShapeDtypeStruct((Bh, n, 3), jnp.int32),
                jax.ShapeDtypeStruct((Bh, n, 3), jnp.float32),
            ],
        )(u_h, k_h)

    interps = []
    for h in range(nh):
        sl = slice(h * Bh, (h + 1) * Bh)
        idx_h, w_h = knn_half(unknown[sl], known[sl])
        interps.append(sc_call(idx_h, w_h, kf4[sl]))
    interp_a = interps[0]
    interp_b = interps[-1]

    vec = pl.BlockSpec((Co, 1), lambda p, b, t: (0, 0))
    out = pl.pallas_call(
        functools.partial(_mlp, NT=NT, N=N, Bh=Bh),
        grid=(3, B, NT),
        in_specs=[
            pl.BlockSpec(
                (1, C2, tile),
                lambda p, b, t: ((p == 0) * jnp.minimum(b, Bh - 1), 0,
                                 (p == 0) * t)),
            pl.BlockSpec(
                (1, C2, tile),
                lambda p, b, t: ((p == 0) * jnp.maximum(b - Bh, 0), 0,
                                 (p == 0) * t)),
            pl.BlockSpec((1, C1, tile),
                         lambda p, b, t: ((p == 0) * b, 0, (p == 0) * t)),
            pl.BlockSpec((Co, Cin), lambda p, b, t: (0, 0)),
            vec,
            pl.BlockSpec((Co, Co), lambda p, b, t: (0, 0)),
            vec, vec, vec, vec, vec,
        ],
        out_specs=pl.BlockSpec((1, Co, tile),
                               lambda p, b, t: ((p == 2) * b, 0, (p == 2) * t)),
        out_shape=jax.ShapeDtypeStruct((B, Co, n), jnp.float32),
        scratch_shapes=[
            pltpu.VMEM((Co, B * n), jnp.float32),
            pltpu.VMEM((Co, B * n), jnp.float32),
            pltpu.VMEM((Co, 1), jnp.float32),
            pltpu.VMEM((Co, 1), jnp.float32),
            pltpu.VMEM((Co, 1), jnp.float32),
            pltpu.VMEM((Co, 1), jnp.float32),
        ],
    )(interp_a, interp_b, unknow_feats, W1, b1c, W2, b2c,
      g1c, be1c, g2c, be2c)

    return out


# speedup vs baseline: 1.4059x; 1.2133x over previous
import functools

import jax
import jax.numpy as jnp
from jax import lax
from jax.experimental import pallas as pl
from jax.experimental.pallas import tpu as pltpu
from jax.experimental.pallas import tpu_sc as plsc

_TILE = 512


def _knn(unknown_ref, known_ref, idx_ref, w_ref):
    U = unknown_ref[0]
    K = known_ref[0]
    su2 = jnp.sum(U * U, axis=1, keepdims=True)
    sk2 = jnp.sum(K * K, axis=1)[None, :]
    dot = lax.dot_general(U, K, (((1,), (1,)), ((), ())),
                          preferred_element_type=jnp.float32)
    d2 = su2 - 2.0 * dot + sk2
    tile, m = d2.shape
    col = lax.broadcasted_iota(jnp.int32, (tile, m), 1)
    BIG = jnp.float32(3.0e38)
    IBIG = jnp.int32(2147483647)
    m1 = jnp.min(d2, axis=1, keepdims=True)
    i1 = jnp.min(jnp.where(d2 == m1, col, IBIG), axis=1, keepdims=True)
    d2b = jnp.where(d2 <= m1, BIG, d2)
    m2 = jnp.min(d2b, axis=1, keepdims=True)
    i2 = jnp.min(jnp.where(d2b == m2, col, IBIG), axis=1, keepdims=True)
    d2c = jnp.where(d2 <= m2, BIG, d2)
    m3 = jnp.min(d2c, axis=1, keepdims=True)
    i3 = jnp.min(jnp.where(d2c == m3, col, IBIG), axis=1, keepdims=True)
    r1 = 1.0 / (m1 + 1e-8)
    r2 = 1.0 / (m2 + 1e-8)
    r3 = 1.0 / (m3 + 1e-8)
    norm = r1 + r2 + r3
    idx_ref[0] = jnp.concatenate([i1, i2, i3], axis=1)
    w_ref[0] = jnp.concatenate([r1 / norm, r2 / norm, r3 / norm], axis=1)


def _make_sc_interp(B, n, m, C2):
    info = plsc.get_sparse_core_info()
    NC, NS, L = info.num_cores, info.num_subcores, info.num_lanes
    ncc = C2 // L
    PC = 512

    NW = NC * NS
    nps = NW // (B * ncc)
    npts = n // max(nps, 1)

    nchunks = npts // PC
    ngroups = PC // L

    @functools.partial(
        pl.kernel,
        mesh=plsc.VectorSubcoreMesh(core_axis_name="c", subcore_axis_name="s"),
        compiler_params=pltpu.CompilerParams(
            use_tc_tiling_on_sc=False, needs_layout_passes=False),
        out_type=jax.ShapeDtypeStruct((B, C2, n), jnp.float32),
        scratch_types=[
            pltpu.VMEM((PC, 3), jnp.int32),
            pltpu.VMEM((PC, 3), jnp.int32),
            pltpu.VMEM((PC, 3), jnp.float32),
            pltpu.VMEM((PC, 3), jnp.float32),
            pltpu.VMEM((m, L), jnp.float32),
            pltpu.VMEM((L, PC), jnp.float32),
            pltpu.VMEM((L, PC), jnp.float32),
            pltpu.SemaphoreType.DMA,
            pltpu.SemaphoreType.DMA,
            pltpu.SemaphoreType.DMA,
            pltpu.SemaphoreType.DMA,
        ],
    )
    def sc_interp(idx_hbm, w_hbm, kf_hbm, out_hbm,
                  ic0, ic1, wc0, wc1, tab_v, ob0, ob1,
                  si0, si1, so0, so1):
        wid = lax.axis_index("s") * NC + lax.axis_index("c")
        b = wid // (ncc * nps)
        cc = (wid // nps) % ncc
        ps = wid % nps
        p0 = ps * npts
        pltpu.sync_copy(kf_hbm.at[b, cc], tab_v)
        iota = lax.iota(jnp.int32, L)
        bufs = [(ic0, wc0, ob0, si0, so0), (ic1, wc1, ob1, si1, so1)]

        def in_descs(ch, ic, wc, si):
            src = pl.ds(p0 + ch * PC, PC)
            return (pltpu.make_async_copy(idx_hbm.at[b, src], ic, si),
                    pltpu.make_async_copy(w_hbm.at[b, src], wc, si))

        def out_desc(ch, ob, so):
            return pltpu.make_async_copy(
                ob, out_hbm.at[b, pl.ds(cc * L, L), pl.ds(p0 + ch * PC, PC)],
                so)

        for d in in_descs(0, *bufs[0][:2], bufs[0][3]):
            d.start()
        for ch in range(nchunks):
            ic, wc, ob, si, so = bufs[ch % 2]
            if ch + 1 < nchunks:
                nic, nwc, _, nsi, _ = bufs[(ch + 1) % 2]
                for d in in_descs(ch + 1, nic, nwc, nsi):
                    d.start()
            for d in in_descs(ch, ic, wc, si):
                d.wait()
            if ch >= 2:
                out_desc(ch - 2, ob, so).wait()

            def group_body(g2, carry, ic=ic, wc=wc, ob=ob):
                for u in range(2):
                    g = 2 * g2 + u
                    rows = g * L + iota
                    accs = None
                    for k in range(3):
                        kv = jnp.full((L,), k, jnp.int32)
                        i16 = plsc.load_gather(ic, [rows, kv])
                        w16 = plsc.load_gather(wc, [rows, kv])
                        cur = []
                        for c in range(L):
                            cv = jnp.full((L,), c, jnp.int32)
                            g16 = plsc.load_gather(tab_v, [i16, cv])
                            contrib = w16 * g16
                            cur.append(contrib if accs is None
                                       else accs[c] + contrib)
                        accs = cur
                    for c, acc in enumerate(accs):
                        ob[c, pl.ds(g * L, L)] = acc
                return carry

            lax.fori_loop(0, ngroups // 2, group_body, 0)
            out_desc(ch, ob, so).start()
        for ch in range(max(nchunks - 2, 0), nchunks):
            _, _, ob, _, so = bufs[ch % 2]
            out_desc(ch, ob, so).wait()

    return sc_interp


def _bn_scale_shift(s_ref, q_ref, g_ref, be_ref, N):
    mu = s_ref[...] / N
    var = q_ref[...] / N - mu * mu
    a = g_ref[...] * lax.rsqrt(var + 1e-5)
    c = be_ref[...] - a * mu
    return a, c


def _mlp(interpa_ref, interpb_ref, uf_ref, W1_ref, b1_ref, W2_ref, b2_ref,
         g1_ref, be1_ref, g2_ref, be2_ref, out_ref,
         h1_ref, h2_ref, s1_ref, q1_ref, s2_ref, q2_ref, *, NT, N, Bh):
    p = pl.program_id(0)
    b = pl.program_id(1)
    t = pl.program_id(2)
    tile = out_ref.shape[2]
    col0 = (b * NT + t) * tile
    first = jnp.logical_and(b == 0, t == 0)

    @pl.when(p == 0)
    def _phase0():
        interp = jnp.where(b < Bh, interpa_ref[0], interpb_ref[0])
        feat = jnp.concatenate([interp, uf_ref[0]], axis=0)
        h1 = lax.dot_general(W1_ref[...], feat, (((1,), (0,)), ((), ())),
                             preferred_element_type=jnp.float32) + b1_ref[...]
        h1_ref[:, pl.ds(col0, tile)] = h1

        @pl.when(first)
        def _():
            s1_ref[...] = jnp.zeros_like(s1_ref)
            q1_ref[...] = jnp.zeros_like(q1_ref)

        s1_ref[...] += jnp.sum(h1, axis=1, keepdims=True)
        q1_ref[...] += jnp.sum(h1 * h1, axis=1, keepdims=True)

    @pl.when(p == 1)
    def _phase1():
        a1, c1 = _bn_scale_shift(s1_ref, q1_ref, g1_ref, be1_ref, N)
        r = jnp.maximum(a1 * h1_ref[:, pl.ds(col0, tile)] + c1, 0.0)
        h2 = lax.dot_general(W2_ref[...], r, (((1,), (0,)), ((), ())),
                             preferred_element_type=jnp.float32) + b2_ref[...]
        h2_ref[:, pl.ds(col0, tile)] = h2

        @pl.when(first)
        def _():
            s2_ref[...] = jnp.zeros_like(s2_ref)
            q2_ref[...] = jnp.zeros_like(q2_ref)

        s2_ref[...] += jnp.sum(h2, axis=1, keepdims=True)
        q2_ref[...] += jnp.sum(h2 * h2, axis=1, keepdims=True)

    @pl.when(p == 2)
    def _phase2():
        a2, c2 = _bn_scale_shift(s2_ref, q2_ref, g2_ref, be2_ref, N)
        out_ref[0] = jnp.maximum(a2 * h2_ref[:, pl.ds(col0, tile)] + c2, 0.0)


def kernel(unknown, known, unknow_feats, known_feats,
           W1, b1, g1, be1, W2, b2, g2, be2):
    B, n, _ = unknown.shape
    m = known.shape[1]
    C1 = unknow_feats.shape[1]
    C2 = known_feats.shape[1]
    Cin = C1 + C2
    Co = W1.shape[0]
    tile = min(_TILE, n)
    NT = n // tile
    N = float(B * n)

    kfT = jnp.transpose(known_feats, (0, 2, 1))
    b1c, g1c, be1c = b1[:, None], g1[:, None], be1[:, None]
    b2c, g2c, be2c = b2[:, None], g2[:, None], be2[:, None]

    L = plsc.get_sparse_core_info().num_lanes
    ncc = C2 // L
    kf4 = jnp.transpose(known_feats.reshape(B, ncc, L, m), (0, 1, 3, 2))

    nh = 2 if (B % 2 == 0 and B >= 2) else 1
    Bh = B // nh
    sc_call = _make_sc_interp(Bh, n, m, C2)

    def knn_half(u_h, k_h):
        return pl.pallas_call(
            _knn,
            grid=(Bh, NT),
            in_specs=[
                pl.BlockSpec((1, tile, 3), lambda b, t: (b, t, 0)),
                pl.BlockSpec((1, m, 3), lambda b, t: (b, 0, 0)),
            ],
            out_specs=[
                pl.BlockSpec((1, tile, 3), lambda b, t: (b, t, 0)),
                pl.BlockSpec((1, tile, 3), lambda b, t: (b, t, 0)),
            ],
            out_shape=[
                jax.ShapeDtypeStruct((Bh, n, 3), jnp.int32),
                jax.ShapeDtypeStruct((Bh, n, 3), jnp.float32),
            ],
        )(u_h, k_h)

    interps = []
    for h in range(nh):
        sl = slice(h * Bh, (h + 1) * Bh)
        idx_h, w_h = knn_half(unknown[sl], known[sl])
        interps.append(sc_call(idx_h, w_h, kf4[sl]))
    interp_a = interps[0]
    interp_b = interps[-1]

    vec = pl.BlockSpec((Co, 1), lambda p, b, t: (0, 0))
    out = pl.pallas_call(
        functools.partial(_mlp, NT=NT, N=N, Bh=Bh),
        grid=(3, B, NT),
        in_specs=[
            pl.BlockSpec(
                (1, C2, tile),
                lambda p, b, t: ((p == 0) * jnp.minimum(b, Bh - 1), 0,
                                 (p == 0) * t)),
            pl.BlockSpec(
                (1, C2, tile),
                lambda p, b, t: ((p == 0) * jnp.maximum(b - Bh, 0), 0,
                                 (p == 0) * t)),
            pl.BlockSpec((1, C1, tile),
                         lambda p, b, t: ((p == 0) * b, 0, (p == 0) * t)),
            pl.BlockSpec((Co, Cin), lambda p, b, t: (0, 0)),
            vec,
            pl.BlockSpec((Co, Co), lambda p, b, t: (0, 0)),
            vec, vec, vec, vec, vec,
        ],
        out_specs=pl.BlockSpec((1, Co, tile),
                               lambda p, b, t: ((p == 2) * b, 0, (p == 2) * t)),
        out_shape=jax.ShapeDtypeStruct((B, Co, n), jnp.float32),
        scratch_shapes=[
            pltpu.VMEM((Co, B * n), jnp.float32),
            pltpu.VMEM((Co, B * n), jnp.float32),
            pltpu.VMEM((Co, 1), jnp.float32),
            pltpu.VMEM((Co, 1), jnp.float32),
            pltpu.VMEM((Co, 1), jnp.float32),
            pltpu.VMEM((Co, 1), jnp.float32),
        ],
    )(interp_a, interp_b, unknow_feats, W1, b1c, W2, b2c,
      g1c, be1c, g2c, be2c)

    return out


# speedup vs baseline: 1.5497x; 1.1023x over previous
import functools

import jax
import jax.numpy as jnp
from jax import lax
from jax.experimental import pallas as pl
from jax.experimental.pallas import tpu as pltpu
from jax.experimental.pallas import tpu_sc as plsc

_TILE = 1024


def _knn(unknown_ref, known_ref, idx_ref, w_ref):
    U = unknown_ref[0]
    K = known_ref[0]
    su2 = jnp.sum(U * U, axis=1, keepdims=True)
    sk2 = jnp.sum(K * K, axis=1)[None, :]
    dot = lax.dot_general(U, K, (((1,), (1,)), ((), ())),
                          preferred_element_type=jnp.float32)
    d2 = su2 - 2.0 * dot + sk2
    tile, m = d2.shape
    col = lax.broadcasted_iota(jnp.int32, (tile, m), 1)
    BIG = jnp.float32(3.0e38)
    IBIG = jnp.int32(2147483647)
    m1 = jnp.min(d2, axis=1, keepdims=True)
    i1 = jnp.min(jnp.where(d2 == m1, col, IBIG), axis=1, keepdims=True)
    d2b = jnp.where(d2 <= m1, BIG, d2)
    m2 = jnp.min(d2b, axis=1, keepdims=True)
    i2 = jnp.min(jnp.where(d2b == m2, col, IBIG), axis=1, keepdims=True)
    d2c = jnp.where(d2 <= m2, BIG, d2)
    m3 = jnp.min(d2c, axis=1, keepdims=True)
    i3 = jnp.min(jnp.where(d2c == m3, col, IBIG), axis=1, keepdims=True)
    r1 = 1.0 / (m1 + 1e-8)
    r2 = 1.0 / (m2 + 1e-8)
    r3 = 1.0 / (m3 + 1e-8)
    norm = r1 + r2 + r3
    idx_ref[0] = jnp.concatenate([i1, i2, i3], axis=1)
    w_ref[0] = jnp.concatenate([r1 / norm, r2 / norm, r3 / norm], axis=1)


def _make_sc_interp(B, n, m, C2):
    info = plsc.get_sparse_core_info()
    NC, NS, L = info.num_cores, info.num_subcores, info.num_lanes
    ncc = C2 // L
    PC = 512

    NW = NC * NS
    nps = NW // (B * ncc)
    npts = n // max(nps, 1)

    nchunks = npts // PC
    ngroups = PC // L

    @functools.partial(
        pl.kernel,
        mesh=plsc.VectorSubcoreMesh(core_axis_name="c", subcore_axis_name="s"),
        compiler_params=pltpu.CompilerParams(
            use_tc_tiling_on_sc=False, needs_layout_passes=False),
        out_type=jax.ShapeDtypeStruct((B, C2, n), jnp.float32),
        scratch_types=[
            pltpu.VMEM((PC, 3), jnp.int32),
            pltpu.VMEM((PC, 3), jnp.int32),
            pltpu.VMEM((PC, 3), jnp.float32),
            pltpu.VMEM((PC, 3), jnp.float32),
            pltpu.VMEM((m, L), jnp.float32),
            pltpu.VMEM((L, PC), jnp.float32),
            pltpu.VMEM((L, PC), jnp.float32),
            pltpu.SemaphoreType.DMA,
            pltpu.SemaphoreType.DMA,
            pltpu.SemaphoreType.DMA,
            pltpu.SemaphoreType.DMA,
        ],
    )
    def sc_interp(idx_hbm, w_hbm, kf_hbm, out_hbm,
                  ic0, ic1, wc0, wc1, tab_v, ob0, ob1,
                  si0, si1, so0, so1):
        wid = lax.axis_index("s") * NC + lax.axis_index("c")
        b = wid // (ncc * nps)
        cc = (wid // nps) % ncc
        ps = wid % nps
        p0 = ps * npts
        pltpu.sync_copy(kf_hbm.at[b, cc], tab_v)
        iota = lax.iota(jnp.int32, L)
        bufs = [(ic0, wc0, ob0, si0, so0), (ic1, wc1, ob1, si1, so1)]

        def in_descs(ch, ic, wc, si):
            src = pl.ds(p0 + ch * PC, PC)
            return (pltpu.make_async_copy(idx_hbm.at[b, src], ic, si),
                    pltpu.make_async_copy(w_hbm.at[b, src], wc, si))

        def out_desc(ch, ob, so):
            return pltpu.make_async_copy(
                ob, out_hbm.at[b, pl.ds(cc * L, L), pl.ds(p0 + ch * PC, PC)],
                so)

        for d in in_descs(0, *bufs[0][:2], bufs[0][3]):
            d.start()
        for ch in range(nchunks):
            ic, wc, ob, si, so = bufs[ch % 2]
            if ch + 1 < nchunks:
                nic, nwc, _, nsi, _ = bufs[(ch + 1) % 2]
                for d in in_descs(ch + 1, nic, nwc, nsi):
                    d.start()
            for d in in_descs(ch, ic, wc, si):
                d.wait()
            if ch >= 2:
                out_desc(ch - 2, ob, so).wait()

            def group_body(g2, carry, ic=ic, wc=wc, ob=ob):
                for u in range(2):
                    g = 2 * g2 + u
                    rows = g * L + iota
                    accs = None
                    for k in range(3):
                        kv = jnp.full((L,), k, jnp.int32)
                        i16 = plsc.load_gather(ic, [rows, kv])
                        w16 = plsc.load_gather(wc, [rows, kv])
                        cur = []
                        for c in range(L):
                            cv = jnp.full((L,), c, jnp.int32)
                            g16 = plsc.load_gather(tab_v, [i16, cv])
                            contrib = w16 * g16
                            cur.append(contrib if accs is None
                                       else accs[c] + contrib)
                        accs = cur
                    for c, acc in enumerate(accs):
                        ob[c, pl.ds(g * L, L)] = acc
                return carry

            lax.fori_loop(0, ngroups // 2, group_body, 0)
            out_desc(ch, ob, so).start()
        for ch in range(max(nchunks - 2, 0), nchunks):
            _, _, ob, _, so = bufs[ch % 2]
            out_desc(ch, ob, so).wait()

    return sc_interp


def _bn_scale_shift(s_ref, q_ref, g_ref, be_ref, N):
    mu = s_ref[...] / N
    var = q_ref[...] / N - mu * mu
    a = g_ref[...] * lax.rsqrt(var + 1e-5)
    c = be_ref[...] - a * mu
    return a, c


def _mlp(interpa_ref, interpb_ref, uf_ref, W1_ref, b1_ref, W2_ref, b2_ref,
         g1_ref, be1_ref, g2_ref, be2_ref, out_ref,
         h1_ref, h2_ref, s1_ref, q1_ref, s2_ref, q2_ref, *, NT, N, Bh):
    p = pl.program_id(0)
    b = pl.program_id(1)
    t = pl.program_id(2)
    tile = out_ref.shape[2]
    col0 = (b * NT + t) * tile
    first = jnp.logical_and(b == 0, t == 0)

    @pl.when(p == 0)
    def _phase0():
        interp = jnp.where(b < Bh, interpa_ref[0], interpb_ref[0])
        feat = jnp.concatenate([interp, uf_ref[0]], axis=0)
        h1 = lax.dot_general(W1_ref[...], feat, (((1,), (0,)), ((), ())),
                             preferred_element_type=jnp.float32) + b1_ref[...]
        h1_ref[:, pl.ds(col0, tile)] = h1

        @pl.when(first)
        def _():
            s1_ref[...] = jnp.zeros_like(s1_ref)
            q1_ref[...] = jnp.zeros_like(q1_ref)

        s1_ref[...] += jnp.sum(h1, axis=1, keepdims=True)
        q1_ref[...] += jnp.sum(h1 * h1, axis=1, keepdims=True)

    @pl.when(p == 1)
    def _phase1():
        a1, c1 = _bn_scale_shift(s1_ref, q1_ref, g1_ref, be1_ref, N)
        r = jnp.maximum(a1 * h1_ref[:, pl.ds(col0, tile)] + c1, 0.0)
        h2 = lax.dot_general(W2_ref[...], r, (((1,), (0,)), ((), ())),
                             preferred_element_type=jnp.float32) + b2_ref[...]
        h2_ref[:, pl.ds(col0, tile)] = h2

        @pl.when(first)
        def _():
            s2_ref[...] = jnp.zeros_like(s2_ref)
            q2_ref[...] = jnp.zeros_like(q2_ref)

        s2_ref[...] += jnp.sum(h2, axis=1, keepdims=True)
        q2_ref[...] += jnp.sum(h2 * h2, axis=1, keepdims=True)

    @pl.when(p == 2)
    def _phase2():
        a2, c2 = _bn_scale_shift(s2_ref, q2_ref, g2_ref, be2_ref, N)
        out_ref[0] = jnp.maximum(a2 * h2_ref[:, pl.ds(col0, tile)] + c2, 0.0)


def kernel(unknown, known, unknow_feats, known_feats,
           W1, b1, g1, be1, W2, b2, g2, be2):
    B, n, _ = unknown.shape
    m = known.shape[1]
    C1 = unknow_feats.shape[1]
    C2 = known_feats.shape[1]
    Cin = C1 + C2
    Co = W1.shape[0]
    tile = min(_TILE, n)
    NT = n // tile
    N = float(B * n)

    kfT = jnp.transpose(known_feats, (0, 2, 1))
    b1c, g1c, be1c = b1[:, None], g1[:, None], be1[:, None]
    b2c, g2c, be2c = b2[:, None], g2[:, None], be2[:, None]

    L = plsc.get_sparse_core_info().num_lanes
    ncc = C2 // L
    kf4 = jnp.transpose(known_feats.reshape(B, ncc, L, m), (0, 1, 3, 2))

    nh = 2 if (B % 2 == 0 and B >= 2) else 1
    Bh = B // nh
    sc_call = _make_sc_interp(Bh, n, m, C2)

    def knn_half(u_h, k_h):
        return pl.pallas_call(
            _knn,
            grid=(Bh, NT),
            in_specs=[
                pl.BlockSpec((1, tile, 3), lambda b, t: (b, t, 0)),
                pl.BlockSpec((1, m, 3), lambda b, t: (b, 0, 0)),
            ],
            out_specs=[
                pl.BlockSpec((1, tile, 3), lambda b, t: (b, t, 0)),
                pl.BlockSpec((1, tile, 3), lambda b, t: (b, t, 0)),
            ],
            out_shape=[
                jax.ShapeDtypeStruct((Bh, n, 3), jnp.int32),
                jax.ShapeDtypeStruct((Bh, n, 3), jnp.float32),
            ],
        )(u_h, k_h)

    interps = []
    for h in range(nh):
        sl = slice(h * Bh, (h + 1) * Bh)
        idx_h, w_h = knn_half(unknown[sl], known[sl])
        interps.append(sc_call(idx_h, w_h, kf4[sl]))
    interp_a = interps[0]
    interp_b = interps[-1]

    vec = pl.BlockSpec((Co, 1), lambda p, b, t: (0, 0))
    out = pl.pallas_call(
        functools.partial(_mlp, NT=NT, N=N, Bh=Bh),
        grid=(3, B, NT),
        in_specs=[
            pl.BlockSpec(
                (1, C2, tile),
                lambda p, b, t: ((p == 0) * jnp.minimum(b, Bh - 1), 0,
                                 (p == 0) * t)),
            pl.BlockSpec(
                (1, C2, tile),
                lambda p, b, t: ((p == 0) * jnp.maximum(b - Bh, 0), 0,
                                 (p == 0) * t)),
            pl.BlockSpec((1, C1, tile),
                         lambda p, b, t: ((p == 0) * b, 0, (p == 0) * t)),
            pl.BlockSpec((Co, Cin), lambda p, b, t: (0, 0)),
            vec,
            pl.BlockSpec((Co, Co), lambda p, b, t: (0, 0)),
            vec, vec, vec, vec, vec,
        ],
        out_specs=pl.BlockSpec((1, Co, tile),
                               lambda p, b, t: ((p == 2) * b, 0, (p == 2) * t)),
        out_shape=jax.ShapeDtypeStruct((B, Co, n), jnp.float32),
        scratch_shapes=[
            pltpu.VMEM((Co, B * n), jnp.float32),
            pltpu.VMEM((Co, B * n), jnp.float32),
            pltpu.VMEM((Co, 1), jnp.float32),
            pltpu.VMEM((Co, 1), jnp.float32),
            pltpu.VMEM((Co, 1), jnp.float32),
            pltpu.VMEM((Co, 1), jnp.float32),
        ],
    )(interp_a, interp_b, unknow_feats, W1, b1c, W2, b2c,
      g1c, be1c, g2c, be2c)

    return out


# speedup vs baseline: 1.6341x; 1.0545x over previous
import functools

import jax
import jax.numpy as jnp
from jax import lax
from jax.experimental import pallas as pl
from jax.experimental.pallas import tpu as pltpu
from jax.experimental.pallas import tpu_sc as plsc

_TILE = 2048


def _knn(unknown_ref, known_ref, idx_ref, w_ref):
    U = unknown_ref[0]
    K = known_ref[0]
    su2 = jnp.sum(U * U, axis=1, keepdims=True)
    sk2 = jnp.sum(K * K, axis=1)[None, :]
    dot = lax.dot_general(U, K, (((1,), (1,)), ((), ())),
                          preferred_element_type=jnp.float32)
    d2 = su2 - 2.0 * dot + sk2
    tile, m = d2.shape
    col = lax.broadcasted_iota(jnp.int32, (tile, m), 1)
    BIG = jnp.float32(3.0e38)
    IBIG = jnp.int32(2147483647)
    m1 = jnp.min(d2, axis=1, keepdims=True)
    i1 = jnp.min(jnp.where(d2 == m1, col, IBIG), axis=1, keepdims=True)
    d2b = jnp.where(d2 <= m1, BIG, d2)
    m2 = jnp.min(d2b, axis=1, keepdims=True)
    i2 = jnp.min(jnp.where(d2b == m2, col, IBIG), axis=1, keepdims=True)
    d2c = jnp.where(d2 <= m2, BIG, d2)
    m3 = jnp.min(d2c, axis=1, keepdims=True)
    i3 = jnp.min(jnp.where(d2c == m3, col, IBIG), axis=1, keepdims=True)
    r1 = 1.0 / (m1 + 1e-8)
    r2 = 1.0 / (m2 + 1e-8)
    r3 = 1.0 / (m3 + 1e-8)
    norm = r1 + r2 + r3
    idx_ref[0] = jnp.concatenate([i1, i2, i3], axis=1)
    w_ref[0] = jnp.concatenate([r1 / norm, r2 / norm, r3 / norm], axis=1)


def _make_sc_interp(B, n, m, C2):
    info = plsc.get_sparse_core_info()
    NC, NS, L = info.num_cores, info.num_subcores, info.num_lanes
    ncc = C2 // L
    PC = 512

    NW = NC * NS
    nps = NW // (B * ncc)
    npts = n // max(nps, 1)

    nchunks = npts // PC
    ngroups = PC // L

    @functools.partial(
        pl.kernel,
        mesh=plsc.VectorSubcoreMesh(core_axis_name="c", subcore_axis_name="s"),
        compiler_params=pltpu.CompilerParams(
            use_tc_tiling_on_sc=False, needs_layout_passes=False),
        out_type=jax.ShapeDtypeStruct((B, C2, n), jnp.float32),
        scratch_types=[
            pltpu.VMEM((PC, 3), jnp.int32),
            pltpu.VMEM((PC, 3), jnp.int32),
            pltpu.VMEM((PC, 3), jnp.float32),
            pltpu.VMEM((PC, 3), jnp.float32),
            pltpu.VMEM((m, L), jnp.float32),
            pltpu.VMEM((L, PC), jnp.float32),
            pltpu.VMEM((L, PC), jnp.float32),
            pltpu.SemaphoreType.DMA,
            pltpu.SemaphoreType.DMA,
            pltpu.SemaphoreType.DMA,
            pltpu.SemaphoreType.DMA,
        ],
    )
    def sc_interp(idx_hbm, w_hbm, kf_hbm, out_hbm,
                  ic0, ic1, wc0, wc1, tab_v, ob0, ob1,
                  si0, si1, so0, so1):
        wid = lax.axis_index("s") * NC + lax.axis_index("c")
        b = wid // (ncc * nps)
        cc = (wid // nps) % ncc
        ps = wid % nps
        p0 = ps * npts
        pltpu.sync_copy(kf_hbm.at[b, cc], tab_v)
        iota = lax.iota(jnp.int32, L)
        bufs = [(ic0, wc0, ob0, si0, so0), (ic1, wc1, ob1, si1, so1)]

        def in_descs(ch, ic, wc, si):
            src = pl.ds(p0 + ch * PC, PC)
            return (pltpu.make_async_copy(idx_hbm.at[b, src], ic, si),
                    pltpu.make_async_copy(w_hbm.at[b, src], wc, si))

        def out_desc(ch, ob, so):
            return pltpu.make_async_copy(
                ob, out_hbm.at[b, pl.ds(cc * L, L), pl.ds(p0 + ch * PC, PC)],
                so)

        for d in in_descs(0, *bufs[0][:2], bufs[0][3]):
            d.start()
        for ch in range(nchunks):
            ic, wc, ob, si, so = bufs[ch % 2]
            if ch + 1 < nchunks:
                nic, nwc, _, nsi, _ = bufs[(ch + 1) % 2]
                for d in in_descs(ch + 1, nic, nwc, nsi):
                    d.start()
            for d in in_descs(ch, ic, wc, si):
                d.wait()
            if ch >= 2:
                out_desc(ch - 2, ob, so).wait()

            def group_body(g2, carry, ic=ic, wc=wc, ob=ob):
                for u in range(2):
                    g = 2 * g2 + u
                    rows = g * L + iota
                    accs = None
                    for k in range(3):
                        kv = jnp.full((L,), k, jnp.int32)
                        i16 = plsc.load_gather(ic, [rows, kv])
                        w16 = plsc.load_gather(wc, [rows, kv])
                        cur = []
                        for c in range(L):
                            cv = jnp.full((L,), c, jnp.int32)
                            g16 = plsc.load_gather(tab_v, [i16, cv])
                            contrib = w16 * g16
                            cur.append(contrib if accs is None
                                       else accs[c] + contrib)
                        accs = cur
                    for c, acc in enumerate(accs):
                        ob[c, pl.ds(g * L, L)] = acc
                return carry

            lax.fori_loop(0, ngroups // 2, group_body, 0)
            out_desc(ch, ob, so).start()
        for ch in range(max(nchunks - 2, 0), nchunks):
            _, _, ob, _, so = bufs[ch % 2]
            out_desc(ch, ob, so).wait()

    return sc_interp


def _bn_scale_shift(s_ref, q_ref, g_ref, be_ref, N):
    mu = s_ref[...] / N
    var = q_ref[...] / N - mu * mu
    a = g_ref[...] * lax.rsqrt(var + 1e-5)
    c = be_ref[...] - a * mu
    return a, c


def _mlp(interpa_ref, interpb_ref, uf_ref, W1_ref, b1_ref, W2_ref, b2_ref,
         g1_ref, be1_ref, g2_ref, be2_ref, out_ref,
         h1_ref, h2_ref, s1_ref, q1_ref, s2_ref, q2_ref, *, NT, N, Bh):
    p = pl.program_id(0)
    b = pl.program_id(1)
    t = pl.program_id(2)
    tile = out_ref.shape[2]
    col0 = (b * NT + t) * tile
    first = jnp.logical_and(b == 0, t == 0)

    @pl.when(p == 0)
    def _phase0():
        interp = jnp.where(b < Bh, interpa_ref[0], interpb_ref[0])
        feat = jnp.concatenate([interp, uf_ref[0]], axis=0)
        h1 = lax.dot_general(W1_ref[...], feat, (((1,), (0,)), ((), ())),
                             preferred_element_type=jnp.float32) + b1_ref[...]
        h1_ref[:, pl.ds(col0, tile)] = h1

        @pl.when(first)
        def _():
            s1_ref[...] = jnp.zeros_like(s1_ref)
            q1_ref[...] = jnp.zeros_like(q1_ref)

        s1_ref[...] += jnp.sum(h1, axis=1, keepdims=True)
        q1_ref[...] += jnp.sum(h1 * h1, axis=1, keepdims=True)

    @pl.when(p == 1)
    def _phase1():
        a1, c1 = _bn_scale_shift(s1_ref, q1_ref, g1_ref, be1_ref, N)
        r = jnp.maximum(a1 * h1_ref[:, pl.ds(col0, tile)] + c1, 0.0)
        h2 = lax.dot_general(W2_ref[...], r, (((1,), (0,)), ((), ())),
                             preferred_element_type=jnp.float32) + b2_ref[...]
        h2_ref[:, pl.ds(col0, tile)] = h2

        @pl.when(first)
        def _():
            s2_ref[...] = jnp.zeros_like(s2_ref)
            q2_ref[...] = jnp.zeros_like(q2_ref)

        s2_ref[...] += jnp.sum(h2, axis=1, keepdims=True)
        q2_ref[...] += jnp.sum(h2 * h2, axis=1, keepdims=True)

    @pl.when(p == 2)
    def _phase2():
        a2, c2 = _bn_scale_shift(s2_ref, q2_ref, g2_ref, be2_ref, N)
        out_ref[0] = jnp.maximum(a2 * h2_ref[:, pl.ds(col0, tile)] + c2, 0.0)


def kernel(unknown, known, unknow_feats, known_feats,
           W1, b1, g1, be1, W2, b2, g2, be2):
    B, n, _ = unknown.shape
    m = known.shape[1]
    C1 = unknow_feats.shape[1]
    C2 = known_feats.shape[1]
    Cin = C1 + C2
    Co = W1.shape[0]
    tile = min(_TILE, n)
    NT = n // tile
    N = float(B * n)

    kfT = jnp.transpose(known_feats, (0, 2, 1))
    b1c, g1c, be1c = b1[:, None], g1[:, None], be1[:, None]
    b2c, g2c, be2c = b2[:, None], g2[:, None], be2[:, None]

    L = plsc.get_sparse_core_info().num_lanes
    ncc = C2 // L
    kf4 = jnp.transpose(known_feats.reshape(B, ncc, L, m), (0, 1, 3, 2))

    nh = 2 if (B % 2 == 0 and B >= 2) else 1
    Bh = B // nh
    sc_call = _make_sc_interp(Bh, n, m, C2)

    def knn_half(u_h, k_h):
        return pl.pallas_call(
            _knn,
            grid=(Bh, NT),
            in_specs=[
                pl.BlockSpec((1, tile, 3), lambda b, t: (b, t, 0)),
                pl.BlockSpec((1, m, 3), lambda b, t: (b, 0, 0)),
            ],
            out_specs=[
                pl.BlockSpec((1, tile, 3), lambda b, t: (b, t, 0)),
                pl.BlockSpec((1, tile, 3), lambda b, t: (b, t, 0)),
            ],
            out_shape=[
                jax.ShapeDtypeStruct((Bh, n, 3), jnp.int32),
                jax.ShapeDtypeStruct((Bh, n, 3), jnp.float32),
            ],
        )(u_h, k_h)

    interps = []
    for h in range(nh):
        sl = slice(h * Bh, (h + 1) * Bh)
        idx_h, w_h = knn_half(unknown[sl], known[sl])
        interps.append(sc_call(idx_h, w_h, kf4[sl]))
    interp_a = interps[0]
    interp_b = interps[-1]

    vec = pl.BlockSpec((Co, 1), lambda p, b, t: (0, 0))
    out = pl.pallas_call(
        functools.partial(_mlp, NT=NT, N=N, Bh=Bh),
        grid=(3, B, NT),
        in_specs=[
            pl.BlockSpec(
                (1, C2, tile),
                lambda p, b, t: ((p == 0) * jnp.minimum(b, Bh - 1), 0,
                                 (p == 0) * t)),
            pl.BlockSpec(
                (1, C2, tile),
                lambda p, b, t: ((p == 0) * jnp.maximum(b - Bh, 0), 0,
                                 (p == 0) * t)),
            pl.BlockSpec((1, C1, tile),
                         lambda p, b, t: ((p == 0) * b, 0, (p == 0) * t)),
            pl.BlockSpec((Co, Cin), lambda p, b, t: (0, 0)),
            vec,
            pl.BlockSpec((Co, Co), lambda p, b, t: (0, 0)),
            vec, vec, vec, vec, vec,
        ],
        out_specs=pl.BlockSpec((1, Co, tile),
                               lambda p, b, t: ((p == 2) * b, 0, (p == 2) * t)),
        out_shape=jax.ShapeDtypeStruct((B, Co, n), jnp.float32),
        scratch_shapes=[
            pltpu.VMEM((Co, B * n), jnp.float32),
            pltpu.VMEM((Co, B * n), jnp.float32),
            pltpu.VMEM((Co, 1), jnp.float32),
            pltpu.VMEM((Co, 1), jnp.float32),
            pltpu.VMEM((Co, 1), jnp.float32),
            pltpu.VMEM((Co, 1), jnp.float32),
        ],
    )(interp_a, interp_b, unknow_feats, W1, b1c, W2, b2c,
      g1c, be1c, g2c, be2c)

    return out


# speedup vs baseline: 1.6795x; 1.0277x over previous
import functools

import jax
import jax.numpy as jnp
from jax import lax
from jax.experimental import pallas as pl
from jax.experimental.pallas import tpu as pltpu
from jax.experimental.pallas import tpu_sc as plsc

_TILE = 2048


def _knn(unknown_ref, known_ref, idx_ref, w_ref):
    U = unknown_ref[0]
    K = known_ref[0]
    su2 = jnp.sum(U * U, axis=1, keepdims=True)
    sk2 = jnp.sum(K * K, axis=1)[None, :]
    dot = lax.dot_general(U, K, (((1,), (1,)), ((), ())),
                          preferred_element_type=jnp.float32)
    d2 = su2 - 2.0 * dot + sk2
    tile, m = d2.shape
    col = lax.broadcasted_iota(jnp.int32, (tile, m), 1)
    BIG = jnp.float32(3.0e38)
    IBIG = jnp.int32(2147483647)
    m1 = jnp.min(d2, axis=1, keepdims=True)
    i1 = jnp.min(jnp.where(d2 == m1, col, IBIG), axis=1, keepdims=True)
    d2b = jnp.where(d2 <= m1, BIG, d2)
    m2 = jnp.min(d2b, axis=1, keepdims=True)
    i2 = jnp.min(jnp.where(d2b == m2, col, IBIG), axis=1, keepdims=True)
    d2c = jnp.where(d2 <= m2, BIG, d2)
    m3 = jnp.min(d2c, axis=1, keepdims=True)
    i3 = jnp.min(jnp.where(d2c == m3, col, IBIG), axis=1, keepdims=True)
    r1 = 1.0 / (m1 + 1e-8)
    r2 = 1.0 / (m2 + 1e-8)
    r3 = 1.0 / (m3 + 1e-8)
    norm = r1 + r2 + r3
    idx_ref[0] = jnp.concatenate([i1, i2, i3], axis=1)
    w_ref[0] = jnp.concatenate([r1 / norm, r2 / norm, r3 / norm], axis=1)


def _make_sc_interp(B, n, m, C2):
    info = plsc.get_sparse_core_info()
    NC, NS, L = info.num_cores, info.num_subcores, info.num_lanes
    ncc = C2 // L
    PC = 512

    NW = NC * NS
    nps = NW // (B * ncc)
    npts = n // max(nps, 1)

    nchunks = npts // PC
    ngroups = PC // L

    @functools.partial(
        pl.kernel,
        mesh=plsc.VectorSubcoreMesh(core_axis_name="c", subcore_axis_name="s"),
        compiler_params=pltpu.CompilerParams(
            use_tc_tiling_on_sc=False, needs_layout_passes=False),
        out_type=jax.ShapeDtypeStruct((B, C2, n), jnp.float32),
        scratch_types=[
            pltpu.VMEM((PC, 3), jnp.int32),
            pltpu.VMEM((PC, 3), jnp.int32),
            pltpu.VMEM((PC, 3), jnp.float32),
            pltpu.VMEM((PC, 3), jnp.float32),
            pltpu.VMEM((m, L), jnp.float32),
            pltpu.VMEM((L, PC), jnp.float32),
            pltpu.VMEM((L, PC), jnp.float32),
            pltpu.SemaphoreType.DMA,
            pltpu.SemaphoreType.DMA,
            pltpu.SemaphoreType.DMA,
            pltpu.SemaphoreType.DMA,
        ],
    )
    def sc_interp(idx_hbm, w_hbm, kf_hbm, out_hbm,
                  ic0, ic1, wc0, wc1, tab_v, ob0, ob1,
                  si0, si1, so0, so1):
        wid = lax.axis_index("s") * NC + lax.axis_index("c")
        b = wid // (ncc * nps)
        cc = (wid // nps) % ncc
        ps = wid % nps
        p0 = ps * npts
        pltpu.sync_copy(kf_hbm.at[b, cc], tab_v)
        iota = lax.iota(jnp.int32, L)
        bufs = [(ic0, wc0, ob0, si0, so0), (ic1, wc1, ob1, si1, so1)]

        def in_descs(ch, ic, wc, si):
            src = pl.ds(p0 + ch * PC, PC)
            return (pltpu.make_async_copy(idx_hbm.at[b, src], ic, si),
                    pltpu.make_async_copy(w_hbm.at[b, src], wc, si))

        def out_desc(ch, ob, so):
            return pltpu.make_async_copy(
                ob, out_hbm.at[b, pl.ds(cc * L, L), pl.ds(p0 + ch * PC, PC)],
                so)

        for d in in_descs(0, *bufs[0][:2], bufs[0][3]):
            d.start()
        for ch in range(nchunks):
            ic, wc, ob, si, so = bufs[ch % 2]
            if ch + 1 < nchunks:
                nic, nwc, _, nsi, _ = bufs[(ch + 1) % 2]
                for d in in_descs(ch + 1, nic, nwc, nsi):
                    d.start()
            for d in in_descs(ch, ic, wc, si):
                d.wait()
            if ch >= 2:
                out_desc(ch - 2, ob, so).wait()

            def group_body(g2, carry, ic=ic, wc=wc, ob=ob):
                for u in range(2):
                    g = 2 * g2 + u
                    rows = g * L + iota
                    accs = None
                    for k in range(3):
                        kv = jnp.full((L,), k, jnp.int32)
                        i16 = plsc.load_gather(ic, [rows, kv])
                        w16 = plsc.load_gather(wc, [rows, kv])
                        cur = []
                        for c in range(L):
                            cv = jnp.full((L,), c, jnp.int32)
                            g16 = plsc.load_gather(tab_v, [i16, cv])
                            contrib = w16 * g16
                            cur.append(contrib if accs is None
                                       else accs[c] + contrib)
                        accs = cur
                    for c, acc in enumerate(accs):
                        ob[c, pl.ds(g * L, L)] = acc
                return carry

            lax.fori_loop(0, ngroups // 2, group_body, 0)
            out_desc(ch, ob, so).start()
        for ch in range(max(nchunks - 2, 0), nchunks):
            _, _, ob, _, so = bufs[ch % 2]
            out_desc(ch, ob, so).wait()

    return sc_interp


def _bn_scale_shift(s_ref, q_ref, g_ref, be_ref, N):
    mu = s_ref[...] / N
    var = q_ref[...] / N - mu * mu
    a = g_ref[...] * lax.rsqrt(var + 1e-5)
    c = be_ref[...] - a * mu
    return a, c


def _mlp(interpa_ref, interpb_ref, uf_ref, W1_ref, b1_ref, W2_ref, b2_ref,
         g1_ref, be1_ref, g2_ref, be2_ref, out_ref,
         h1_ref, h2_ref, s1_ref, q1_ref, s2_ref, q2_ref, *, NT, N, Bh):
    p = pl.program_id(0)
    b = pl.program_id(1)
    t = pl.program_id(2)
    tile = out_ref.shape[2]
    col0 = (b * NT + t) * tile
    first = jnp.logical_and(b == 0, t == 0)

    @pl.when(p == 0)
    def _phase0():
        interp = jnp.where(b < Bh, interpa_ref[0], interpb_ref[0])
        feat = jnp.concatenate([interp, uf_ref[0]], axis=0)
        h1 = lax.dot_general(W1_ref[...], feat, (((1,), (0,)), ((), ())),
                             preferred_element_type=jnp.float32) + b1_ref[...]
        h1_ref[:, pl.ds(col0, tile)] = h1

        @pl.when(first)
        def _():
            s1_ref[...] = jnp.zeros_like(s1_ref)
            q1_ref[...] = jnp.zeros_like(q1_ref)

        s1_ref[...] += jnp.sum(h1, axis=1, keepdims=True)
        q1_ref[...] += jnp.sum(h1 * h1, axis=1, keepdims=True)

    @pl.when(p == 1)
    def _phase1():
        a1, c1 = _bn_scale_shift(s1_ref, q1_ref, g1_ref, be1_ref, N)
        r = jnp.maximum(a1 * h1_ref[:, pl.ds(col0, tile)] + c1, 0.0)
        h2 = lax.dot_general(W2_ref[...], r, (((1,), (0,)), ((), ())),
                             preferred_element_type=jnp.float32) + b2_ref[...]
        h2_ref[:, pl.ds(col0, tile)] = h2

        @pl.when(first)
        def _():
            s2_ref[...] = jnp.zeros_like(s2_ref)
            q2_ref[...] = jnp.zeros_like(q2_ref)

        s2_ref[...] += jnp.sum(h2, axis=1, keepdims=True)
        q2_ref[...] += jnp.sum(h2 * h2, axis=1, keepdims=True)

    @pl.when(p == 2)
    def _phase2():
        a2, c2 = _bn_scale_shift(s2_ref, q2_ref, g2_ref, be2_ref, N)
        out_ref[0] = jnp.maximum(a2 * h2_ref[:, pl.ds(col0, tile)] + c2, 0.0)


def kernel(unknown, known, unknow_feats, known_feats,
           W1, b1, g1, be1, W2, b2, g2, be2):
    B, n, _ = unknown.shape
    m = known.shape[1]
    C1 = unknow_feats.shape[1]
    C2 = known_feats.shape[1]
    Cin = C1 + C2
    Co = W1.shape[0]
    tile = min(_TILE, n)
    NT = n // tile
    N = float(B * n)

    kfT = jnp.transpose(known_feats, (0, 2, 1))
    b1c, g1c, be1c = b1[:, None], g1[:, None], be1[:, None]
    b2c, g2c, be2c = b2[:, None], g2[:, None], be2[:, None]

    L = plsc.get_sparse_core_info().num_lanes
    ncc = C2 // L
    kf4 = jnp.transpose(known_feats.reshape(B, ncc, L, m), (0, 1, 3, 2))

    nh = 2 if (B % 2 == 0 and B >= 2) else 1
    Bh = B // nh
    sc_call = _make_sc_interp(Bh, n, m, C2)

    def knn_half(u_h, k_h):
        return pl.pallas_call(
            _knn,
            grid=(Bh, NT),
            in_specs=[
                pl.BlockSpec((1, tile, 3), lambda b, t: (b, t, 0)),
                pl.BlockSpec((1, m, 3), lambda b, t: (b, 0, 0)),
            ],
            out_specs=[
                pl.BlockSpec((1, tile, 3), lambda b, t: (b, t, 0)),
                pl.BlockSpec((1, tile, 3), lambda b, t: (b, t, 0)),
            ],
            out_shape=[
                jax.ShapeDtypeStruct((Bh, n, 3), jnp.int32),
                jax.ShapeDtypeStruct((Bh, n, 3), jnp.float32),
            ],
        )(u_h, k_h)

    interps = []
    for h in range(nh):
        sl = slice(h * Bh, (h + 1) * Bh)
        idx_h, w_h = knn_half(unknown[sl], known[sl])
        interps.append(sc_call(idx_h, w_h, kf4[sl]))
    interp_a = interps[0]
    interp_b = interps[-1]

    tile_m = min(4096, n)
    NT_m = n // tile_m
    vec = pl.BlockSpec((Co, 1), lambda p, b, t: (0, 0))
    out = pl.pallas_call(
        functools.partial(_mlp, NT=NT_m, N=N, Bh=Bh),
        grid=(3, B, NT_m),
        in_specs=[
            pl.BlockSpec(
                (1, C2, tile_m),
                lambda p, b, t: ((p == 0) * jnp.minimum(b, Bh - 1), 0,
                                 (p == 0) * t)),
            pl.BlockSpec(
                (1, C2, tile_m),
                lambda p, b, t: ((p == 0) * jnp.maximum(b - Bh, 0), 0,
                                 (p == 0) * t)),
            pl.BlockSpec((1, C1, tile_m),
                         lambda p, b, t: ((p == 0) * b, 0, (p == 0) * t)),
            pl.BlockSpec((Co, Cin), lambda p, b, t: (0, 0)),
            vec,
            pl.BlockSpec((Co, Co), lambda p, b, t: (0, 0)),
            vec, vec, vec, vec, vec,
        ],
        out_specs=pl.BlockSpec(
            (1, Co, tile_m),
            lambda p, b, t: ((p == 2) * b, 0, (p == 2) * t)),
        out_shape=jax.ShapeDtypeStruct((B, Co, n), jnp.float32),
        scratch_shapes=[
            pltpu.VMEM((Co, B * n), jnp.float32),
            pltpu.VMEM((Co, B * n), jnp.float32),
            pltpu.VMEM((Co, 1), jnp.float32),
            pltpu.VMEM((Co, 1), jnp.float32),
            pltpu.VMEM((Co, 1), jnp.float32),
            pltpu.VMEM((Co, 1), jnp.float32),
        ],
    )(interp_a, interp_b, unknow_feats, W1, b1c, W2, b2c,
      g1c, be1c, g2c, be2c)

    return out
